# baseline (speedup 1.0000x reference)
import functools

import jax
import jax.numpy as jnp
from jax import lax
from jax.experimental import pallas as pl
from jax.experimental.pallas import tpu as pltpu
from jax.experimental.pallas import tpu_sc as plsc

NUM_CLASSES = 100000
FEAT = 128
BATCH = 16384
NC = 2
NS = 16
L = 16
NW = NC * NS
ROWS_PER_W = BATCH // NW
CHUNK = 128
NCHUNK = ROWS_PER_W // CHUNK
CGROUPS = FEAT // L

_mesh = plsc.VectorSubcoreMesh(core_axis_name="c", subcore_axis_name="s")


@functools.partial(
    pl.kernel,
    mesh=_mesh,
    out_type=jax.ShapeDtypeStruct((NW, L), jnp.float32),
    scratch_types=[
        pltpu.VMEM((NCHUNK, CHUNK), jnp.int32),
        pltpu.VMEM((2, CHUNK, FEAT), jnp.float32),
        pltpu.VMEM((2, CHUNK, FEAT), jnp.float32),
        pltpu.VMEM((L,), jnp.float32),
        pltpu.SemaphoreType.DMA,
        pltpu.SemaphoreType.DMA,
        pltpu.SemaphoreType.DMA,
        pltpu.SemaphoreType.DMA,
    ],
)
def _center_loss_sc(feat_hbm, lab_hbm, cent_hbm, out_hbm,
                    idx_v, fbuf, cbuf, acc_v, fsem0, fsem1, gsem0, gsem1):
    wid = lax.axis_index("s") * NC + lax.axis_index("c")
    base = wid * ROWS_PER_W

    pltpu.sync_copy(lab_hbm.at[wid], idx_v)

    fsems = (fsem0, fsem1)
    gsems = (gsem0, gsem1)

    def start(j, slot):
        fcp = pltpu.async_copy(
            feat_hbm.at[pl.ds(base + j * CHUNK, CHUNK)], fbuf.at[slot],
            fsems[slot])
        gcp = pltpu.async_copy(cent_hbm.at[idx_v.at[j]], cbuf.at[slot],
                               gsems[slot])
        return fcp, gcp

    UNROLL = 2

    def chunk_sum(slot, accs):
        f_ref = fbuf.at[slot]
        c_ref = cbuf.at[slot]

        def body(i, accs):
            r = i * UNROLL
            out = list(accs)
            for u in range(UNROLL):
                for k in range(CGROUPS):
                    f = f_ref[r + u, pl.ds(k * L, L)]
                    c = c_ref[r + u, pl.ds(k * L, L)]
                    d = f - c
                    out[k] = out[k] + d * d
            return tuple(out)
        return lax.fori_loop(0, CHUNK // UNROLL, body, accs)

    accs = tuple(jnp.zeros((L,), jnp.float32) for _ in range(CGROUPS))
    pending = start(0, 0)
    for j in range(NCHUNK):
        slot = j % 2
        nxt = start(j + 1, 1 - slot) if j + 1 < NCHUNK else None
        pending[0].wait()
        pending[1].wait()
        accs = chunk_sum(slot, accs)
        pending = nxt

    total = accs[0]
    for k in range(1, CGROUPS):
        total = total + accs[k]
    acc_v[...] = total
    pltpu.sync_copy(acc_v, out_hbm.at[wid])


def kernel(features, labels, centers):
    lab = labels.astype(jnp.int32).reshape(NW, NCHUNK, CHUNK)
    partials = _center_loss_sc(features, lab, centers)
    return jnp.sum(partials) * 0.5


# speedup vs baseline: 1.0147x; 1.0147x over previous
import functools

import jax
import jax.numpy as jnp
from jax import lax
from jax.experimental import pallas as pl
from jax.experimental.pallas import tpu as pltpu
from jax.experimental.pallas import tpu_sc as plsc

NUM_CLASSES = 100000
FEAT = 128
BATCH = 16384
NC = 2
NS = 16
L = 16
NW = NC * NS
ROWS_PER_W = BATCH // NW
CHUNK = 128
NCHUNK = ROWS_PER_W // CHUNK
CGROUPS = FEAT // L

_mesh = plsc.VectorSubcoreMesh(core_axis_name="c", subcore_axis_name="s")


@functools.partial(
    pl.kernel,
    mesh=_mesh,
    out_type=jax.ShapeDtypeStruct((NW, L), jnp.float32),
    scratch_types=[
        pltpu.VMEM((NCHUNK, CHUNK), jnp.int32),
        pltpu.VMEM((2, CHUNK, FEAT), jnp.float32),
        pltpu.VMEM((2, CHUNK, FEAT), jnp.float32),
        pltpu.VMEM((L,), jnp.float32),
        pltpu.SemaphoreType.DMA,
        pltpu.SemaphoreType.DMA,
        pltpu.SemaphoreType.DMA,
        pltpu.SemaphoreType.DMA,
    ],
)
def _center_loss_sc(feat_hbm, lab_hbm, cent_hbm, out_hbm,
                    idx_v, fbuf, cbuf, acc_v, fsem0, fsem1, gsem0, gsem1):
    wid = lax.axis_index("s") * NC + lax.axis_index("c")
    base = wid * ROWS_PER_W

    pltpu.sync_copy(lab_hbm.at[wid], idx_v)

    fsems = (fsem0, fsem1)
    gsems = (gsem0, gsem1)

    def start(j, slot):
        fcp = pltpu.async_copy(
            feat_hbm.at[pl.ds(base + j * CHUNK, CHUNK)], fbuf.at[slot],
            fsems[slot])
        gcp = pltpu.async_copy(cent_hbm.at[idx_v.at[j]], cbuf.at[slot],
                               gsems[slot])
        return fcp, gcp

    def chunk_sum(slot, accs):
        f_ref = fbuf.at[slot]
        c_ref = cbuf.at[slot]

        def body(r, accs):
            out = list(accs)
            for k in range(CGROUPS):
                f = f_ref[r, pl.ds(k * L, L)]
                c = c_ref[r, pl.ds(k * L, L)]
                d = f - c
                out[k] = out[k] + d * d
            return tuple(out)
        return plsc.parallel_loop(0, CHUNK, 1, unroll=4, carry=accs)(body)

    accs = tuple(jnp.zeros((L,), jnp.float32) for _ in range(CGROUPS))
    pending = start(0, 0)
    for j in range(NCHUNK):
        slot = j % 2
        nxt = start(j + 1, 1 - slot) if j + 1 < NCHUNK else None
        pending[0].wait()
        pending[1].wait()
        accs = chunk_sum(slot, accs)
        pending = nxt

    total = accs[0]
    for k in range(1, CGROUPS):
        total = total + accs[k]
    acc_v[...] = total
    pltpu.sync_copy(acc_v, out_hbm.at[wid])


def kernel(features, labels, centers):
    lab = labels.astype(jnp.int32).reshape(NW, NCHUNK, CHUNK)
    partials = _center_loss_sc(features, lab, centers)
    return jnp.sum(partials) * 0.5


# speedup vs baseline: 1.0451x; 1.0299x over previous
import functools

import jax
import jax.numpy as jnp
from jax import lax
from jax.experimental import pallas as pl
from jax.experimental.pallas import tpu as pltpu
from jax.experimental.pallas import tpu_sc as plsc

NUM_CLASSES = 100000
FEAT = 128
BATCH = 16384
NC = 2
NS = 16
L = 16
NW = NC * NS
ROWS_PER_W = BATCH // NW
CHUNK = 64
NCHUNK = ROWS_PER_W // CHUNK
NSLOT = 3
CGROUPS = FEAT // L

_mesh = plsc.VectorSubcoreMesh(core_axis_name="c", subcore_axis_name="s")


@functools.partial(
    pl.kernel,
    mesh=_mesh,
    out_type=jax.ShapeDtypeStruct((NW, L), jnp.float32),
    scratch_types=[
        pltpu.VMEM((NCHUNK, CHUNK), jnp.int32),
        pltpu.VMEM((NSLOT, CHUNK, FEAT), jnp.float32),
        pltpu.VMEM((NSLOT, CHUNK, FEAT), jnp.float32),
        pltpu.VMEM((L,), jnp.float32),
        pltpu.SemaphoreType.DMA,
        pltpu.SemaphoreType.DMA,
        pltpu.SemaphoreType.DMA,
        pltpu.SemaphoreType.DMA,
        pltpu.SemaphoreType.DMA,
        pltpu.SemaphoreType.DMA,
    ],
)
def _center_loss_sc(feat_hbm, lab_hbm, cent_hbm, out_hbm,
                    idx_v, fbuf, cbuf, acc_v,
                    fsem0, fsem1, fsem2, gsem0, gsem1, gsem2):
    wid = lax.axis_index("s") * NC + lax.axis_index("c")
    base = wid * ROWS_PER_W

    fsems = (fsem0, fsem1, fsem2)
    gsems = (gsem0, gsem1, gsem2)

    def start_f(j, slot):
        return pltpu.async_copy(
            feat_hbm.at[pl.ds(base + j * CHUNK, CHUNK)], fbuf.at[slot],
            fsems[slot])

    def start_g(j, slot):
        return pltpu.async_copy(cent_hbm.at[idx_v.at[j]], cbuf.at[slot],
                                gsems[slot])

    def chunk_sum(slot, accs):
        f_ref = fbuf.at[slot]
        c_ref = cbuf.at[slot]

        def body(r, accs):
            out = list(accs)
            for k in range(CGROUPS):
                f = f_ref[r, pl.ds(k * L, L)]
                c = c_ref[r, pl.ds(k * L, L)]
                d = f - c
                out[k] = out[k] + d * d
            return tuple(out)
        return plsc.parallel_loop(0, CHUNK, 1, unroll=4, carry=accs)(body)

    fcp = [start_f(j, j) for j in range(NSLOT)]
    pltpu.sync_copy(lab_hbm.at[wid], idx_v)
    gcp = [start_g(j, j) for j in range(NSLOT)]

    accs = tuple(jnp.zeros((L,), jnp.float32) for _ in range(CGROUPS))
    for j in range(NCHUNK):
        slot = j % NSLOT
        fcp[slot].wait()
        gcp[slot].wait()
        accs = chunk_sum(slot, accs)
        if j + NSLOT < NCHUNK:
            fcp[slot] = start_f(j + NSLOT, slot)
            gcp[slot] = start_g(j + NSLOT, slot)

    total = accs[0]
    for k in range(1, CGROUPS):
        total = total + accs[k]
    acc_v[...] = total
    pltpu.sync_copy(acc_v, out_hbm.at[wid])


def kernel(features, labels, centers):
    lab = labels.astype(jnp.int32).reshape(NW, NCHUNK, CHUNK)
    partials = _center_loss_sc(features, lab, centers)
    return jnp.sum(partials) * 0.5
